# baseline (speedup 1.0000x reference)
import functools
import math

import jax
import jax.numpy as jnp
from jax import lax
from jax.experimental import pallas as pl
from jax.experimental.pallas import tpu as pltpu
from jax.experimental.pallas import tpu_sc as plsc

_N = 10000
_E = 320000
_D = 128
_A = 8
_S = 16
_H = 64

_NC = 2
_NS = 16
_NW = _NC * _NS
_EPW = _E // _NW
_C = 40
_NCHUNK = _EPW // _C
_KB = 50
_NB = _NCHUNK // _KB
_NPAIR = _NCHUNK // 2
_NP = 10240
_RPS = _NP // _NS

_INV_FAN = 1.0 / math.sqrt(float(_D * _A))


def _fctp_body(x_ref, a_ref, w_ref, o_ref):
    x = x_ref[...]
    a = a_ref[...]
    acc = jnp.zeros(x.shape, jnp.float32)
    for j in range(_A):
        acc += a[:, j:j + 1] * jnp.dot(x, w_ref[j], preferred_element_type=jnp.float32)
    o_ref[...] = acc * _INV_FAN


def _fctp(x, a, w_t):
    bn = 2000
    return pl.pallas_call(
        _fctp_body,
        grid=(_N // bn,),
        in_specs=[
            pl.BlockSpec((bn, _D), lambda i: (i, 0)),
            pl.BlockSpec((bn, _A), lambda i: (i, 0)),
            pl.BlockSpec((_A, _D, _D), lambda i: (0, 0, 0)),
        ],
        out_specs=pl.BlockSpec((bn, _D), lambda i: (i, 0)),
        out_shape=jax.ShapeDtypeStruct((_N, _D), jnp.float32),
    )(x, a, w_t)


_BE = 2000
_CPB = _BE // _C
_BPW = _EPW // _BE


def _edge_mlp_body(es_ref, attr_ref, wfc1_ref, wfc2_ref, out_ref):
    es = es_ref[...]
    h = jnp.dot(es, wfc1_ref[...], preferred_element_type=jnp.float32)
    h = h * (1.0 / math.sqrt(float(_S)))
    h = h * jax.nn.sigmoid(h)
    w = jnp.dot(h, wfc2_ref[...], preferred_element_type=jnp.float32)
    w = w * (1.0 / math.sqrt(float(_H)))
    out_ref[...] = (w * attr_ref[...]).reshape(1, _CPB, _C, _D)


def _edge_mlp(es, attr_scaled, wfc1, wfc2):
    return pl.pallas_call(
        _edge_mlp_body,
        grid=(_NW, _BPW),
        in_specs=[
            pl.BlockSpec((_BE, _S), lambda w, b: (w * _BPW + b, 0)),
            pl.BlockSpec((_BE, 1), lambda w, b: (w * _BPW + b, 0)),
            pl.BlockSpec((_S, _H), lambda w, b: (0, 0)),
            pl.BlockSpec((_H, _D), lambda w, b: (0, 0)),
        ],
        out_specs=pl.BlockSpec((1, _CPB, _C, _D), lambda w, b: (w, b, 0, 0)),
        out_shape=jax.ShapeDtypeStruct((_NW, _NCHUNK, _C, _D), jnp.float32),
    )(es, attr_scaled, wfc1, wfc2)


def _mul_rows(rows, wrow):
    def _mul(e, carry):
        for k in range(_D // 16):
            sl = pl.ds(k * 16, 16)
            rows[e, sl] = rows[e, sl] * wrow[e, sl]
        return carry

    lax.fori_loop(0, _C, _mul, 0)


def _edge_scatter_body(nf_hbm, w_hbm, src_hbm, dst_hbm, out_hbm,
                       sidx_v, didx_v, rows0, rows1, wrow0, wrow1, acc_sh,
                       gs0, gs1, ws0, ws1, ss0, ss1):
    cid = lax.axis_index("c")
    sid = lax.axis_index("s")
    wid = sid * _NC + cid
    sems = (gs0, gs1, ws0, ws1, ss0, ss1)

    zero16 = jnp.zeros((16,), jnp.float32)

    def _zero_row(i, carry):
        for k in range(_D // 16):
            wrow0[i, pl.ds(k * 16, 16)] = zero16
        return carry

    lax.fori_loop(0, _C, _zero_row, 0)
    zdescs = []
    for jj in range(_RPS // _C):
        zdescs.append(pltpu.async_copy(
            wrow0, acc_sh.at[pl.ds(sid * _RPS + jj * _C, _C)], sems[jj % 6]))
    for d in zdescs:
        d.wait()

    pltpu.sync_copy(src_hbm.at[wid, 0], sidx_v)
    pltpu.sync_copy(dst_hbm.at[wid, 0], didx_v)
    pltpu.async_copy(nf_hbm.at[sidx_v.at[0]], rows0, gs0)
    pltpu.async_copy(w_hbm.at[wid, 0], wrow0, ws0)
    pltpu.async_copy(nf_hbm.at[sidx_v.at[1]], rows1, gs1)
    pltpu.async_copy(w_hbm.at[wid, 1], wrow1, ws1)

    plsc.subcore_barrier()

    def _pair(i, carry):
        c0 = 2 * i
        j0 = lax.rem(c0, _KB)
        j1 = j0 + 1
        pltpu.make_async_copy(nf_hbm.at[sidx_v.at[0]], rows0, gs0).wait()
        pltpu.make_async_copy(w_hbm.at[wid, 0], wrow0, ws0).wait()
        _mul_rows(rows0, wrow0)
        pltpu.async_copy(rows0, acc_sh.at[didx_v.at[j0]], ss0, add=True)
        pltpu.make_async_copy(nf_hbm.at[sidx_v.at[1]], rows1, gs1).wait()
        pltpu.make_async_copy(w_hbm.at[wid, 0], wrow1, ws1).wait()
        _mul_rows(rows1, wrow1)
        pltpu.sync_copy(rows1, acc_sh.at[didx_v.at[j1]], add=True)
        pltpu.make_async_copy(rows0, acc_sh.at[didx_v.at[0]], ss0).wait()

        @pl.when(i < _NPAIR - 1)
        def _refill():
            nb = i + 1

            @pl.when(lax.rem(nb, _KB // 2) == 0)
            def _next_batch():
                b = lax.div(nb, _KB // 2)
                pltpu.sync_copy(src_hbm.at[wid, b], sidx_v)
                pltpu.sync_copy(dst_hbm.at[wid, b], didx_v)

            c0n = 2 * nb
            j0n = lax.rem(c0n, _KB)
            pltpu.async_copy(nf_hbm.at[sidx_v.at[j0n]], rows0, gs0)
            pltpu.async_copy(w_hbm.at[wid, c0n], wrow0, ws0)
            pltpu.async_copy(nf_hbm.at[sidx_v.at[j0n + 1]], rows1, gs1)
            pltpu.async_copy(w_hbm.at[wid, c0n + 1], wrow1, ws1)

        return carry

    lax.fori_loop(0, _NPAIR, _pair, 0)
    plsc.subcore_barrier()

    base = sid * _RPS
    pltpu.sync_copy(acc_sh.at[pl.ds(base, _RPS)], out_hbm.at[cid, pl.ds(base, _RPS)])


_edge_scatter = functools.partial(
    pl.kernel,
    out_type=jax.ShapeDtypeStruct((_NC, _NP, _D), jnp.float32),
    mesh=plsc.VectorSubcoreMesh(core_axis_name="c", subcore_axis_name="s"),
    scratch_types=[
        pltpu.VMEM((_KB, _C), jnp.int32),
        pltpu.VMEM((_KB, _C), jnp.int32),
        pltpu.VMEM((_C, _D), jnp.float32),
        pltpu.VMEM((_C, _D), jnp.float32),
        pltpu.VMEM((_C, _D), jnp.float32),
        pltpu.VMEM((_C, _D), jnp.float32),
        pltpu.VMEM_SHARED((_NP, _D), jnp.float32),
        pltpu.SemaphoreType.DMA,
        pltpu.SemaphoreType.DMA,
        pltpu.SemaphoreType.DMA,
        pltpu.SemaphoreType.DMA,
        pltpu.SemaphoreType.DMA,
        pltpu.SemaphoreType.DMA,
    ],
)(_edge_scatter_body)


def _post_body(p0_ref, p1_ref, a_ref, wl2_ref, w3_ref, nsc_ref, out_ref):
    agg = p0_ref[...] + p1_ref[...]
    a = a_ref[...]
    acc = jnp.zeros(agg.shape, jnp.float32)
    for j in range(_A):
        acc += a[:, j:j + 1] * jnp.dot(agg, wl2_ref[j], preferred_element_type=jnp.float32)
    conv = acc * _INV_FAN
    t = jnp.dot(agg, w3_ref[...], preferred_element_type=jnp.float32)
    angle = (0.1 * _INV_FAN) * jnp.sum(t * a, axis=1, keepdims=True)
    out_ref[...] = jnp.cos(angle) * nsc_ref[...] + jnp.sin(angle) * conv


def _post(p0, p1, a, wl2_t, w3_r, nsc):
    bn = 2000
    return pl.pallas_call(
        _post_body,
        grid=(_N // bn,),
        in_specs=[
            pl.BlockSpec((bn, _D), lambda i: (i, 0)),
            pl.BlockSpec((bn, _D), lambda i: (i, 0)),
            pl.BlockSpec((bn, _A), lambda i: (i, 0)),
            pl.BlockSpec((_A, _D, _D), lambda i: (0, 0, 0)),
            pl.BlockSpec((_D, _A), lambda i: (0, 0)),
            pl.BlockSpec((bn, _D), lambda i: (i, 0)),
        ],
        out_specs=pl.BlockSpec((bn, _D), lambda i: (i, 0)),
        out_shape=jax.ShapeDtypeStruct((_N, _D), jnp.float32),
    )(p0, p1, a, wl2_t, w3_r, nsc)


def kernel(node_input, node_attr, edge_src, edge_dst, edge_attr, edge_scalars,
           num_neighbors, W_sc, W_lin1, W_fc1, W_fc2, W_lin2, W_lin3):
    wsc_t = jnp.transpose(W_sc, (1, 0, 2))
    wl1_t = jnp.transpose(W_lin1, (1, 0, 2))
    wl2_t = jnp.transpose(W_lin2, (1, 0, 2))
    w3_r = W_lin3.reshape(_D, _A)

    inv_nb = 1.0 / jnp.sqrt(jnp.asarray(num_neighbors, jnp.float32))
    attr_scaled = edge_attr.astype(jnp.float32) * inv_nb

    nf = _fctp(node_input, node_attr, wl1_t)
    w_scaled = _edge_mlp(edge_scalars, attr_scaled, W_fc1, W_fc2)

    src4 = edge_src.astype(jnp.int32).reshape(_NW, _NB, _KB, _C)
    dst4 = edge_dst.astype(jnp.int32).reshape(_NW, _NB, _KB, _C)

    partials = _edge_scatter(nf, w_scaled, src4, dst4)

    nsc = _fctp(node_input, node_attr, wsc_t)

    return _post(partials[0, :_N], partials[1, :_N], node_attr, wl2_t, w3_r, nsc)


# speedup vs baseline: 1.5195x; 1.5195x over previous
import functools
import math

import jax
import jax.numpy as jnp
from jax import lax
from jax.experimental import pallas as pl
from jax.experimental.pallas import tpu as pltpu
from jax.experimental.pallas import tpu_sc as plsc

_N = 10000
_E = 320000
_D = 128
_A = 8
_S = 16
_H = 64

_NC = 2
_NS = 16
_NW = _NC * _NS
_EPW = _E // _NW
_C = 40
_NCHUNK = _EPW // _C
_KB = 50
_NB = _NCHUNK // _KB
_NPAIR = _NCHUNK // 2
_NP = 10240
_RPS = _NP // _NS

_INV_FAN = 1.0 / math.sqrt(float(_D * _A))


def _fctp_body(x_ref, a_ref, w_ref, o_ref):
    x = x_ref[...]
    a = a_ref[...]
    acc = jnp.zeros(x.shape, jnp.float32)
    for j in range(_A):
        acc += a[:, j:j + 1] * jnp.dot(x, w_ref[j], preferred_element_type=jnp.float32)
    o_ref[...] = acc * _INV_FAN


def _fctp(x, a, w_t):
    bn = 2000
    return pl.pallas_call(
        _fctp_body,
        grid=(_N // bn,),
        in_specs=[
            pl.BlockSpec((bn, _D), lambda i: (i, 0)),
            pl.BlockSpec((bn, _A), lambda i: (i, 0)),
            pl.BlockSpec((_A, _D, _D), lambda i: (0, 0, 0)),
        ],
        out_specs=pl.BlockSpec((bn, _D), lambda i: (i, 0)),
        out_shape=jax.ShapeDtypeStruct((_N, _D), jnp.float32),
    )(x, a, w_t)


def _edge_mlp_body(est_ref, wfc1_ref, wfc2_ref, out_ref):
    est = est_ref[...]
    h = lax.dot_general(est, wfc1_ref[...], (((0,), (0,)), ((), ())),
                        preferred_element_type=jnp.float32)
    h = h * (1.0 / math.sqrt(float(_S)))
    h = h * jax.nn.sigmoid(h)
    out_ref[...] = jnp.dot(h, wfc2_ref[...], preferred_element_type=jnp.float32)


def _edge_mlp(es_t, wfc1, wfc2_scaled):
    be = 3200
    return pl.pallas_call(
        _edge_mlp_body,
        grid=(_E // be,),
        in_specs=[
            pl.BlockSpec((_S, be), lambda i: (0, i)),
            pl.BlockSpec((_S, _H), lambda i: (0, 0)),
            pl.BlockSpec((_H, _D), lambda i: (0, 0)),
        ],
        out_specs=pl.BlockSpec((be, _D), lambda i: (i, 0)),
        out_shape=jax.ShapeDtypeStruct((_E, _D), jnp.float32),
    )(es_t, wfc1, wfc2_scaled)


def _mul_rows(rows, wrow, ax):
    def _mul(e, carry):
        av = ax[pl.ds(e * 16, 16)]
        for k in range(_D // 16):
            sl = pl.ds(k * 16, 16)
            rows[e, sl] = rows[e, sl] * wrow[e, sl] * av
        return carry

    lax.fori_loop(0, _C, _mul, 0)


def _edge_scatter_body(nf_hbm, w_hbm, src_hbm, dst_hbm, attr_hbm, out_hbm,
                       sidx_v, didx_v, ax0, ax1, rows0, rows1, wrow0, wrow1, acc_sh,
                       gs0, gs1, ws0, ws1, ss0, ss1, as0, as1):
    cid = lax.axis_index("c")
    sid = lax.axis_index("s")
    wid = sid * _NC + cid
    sems = (gs0, gs1, ws0, ws1, ss0, ss1)

    zero16 = jnp.zeros((16,), jnp.float32)

    def _zero_row(i, carry):
        for k in range(_D // 16):
            wrow0[i, pl.ds(k * 16, 16)] = zero16
        return carry

    lax.fori_loop(0, _C, _zero_row, 0)
    zdescs = []
    for jj in range(_RPS // _C):
        zdescs.append(pltpu.async_copy(
            wrow0, acc_sh.at[pl.ds(sid * _RPS + jj * _C, _C)], sems[jj % 6]))
    for d in zdescs:
        d.wait()

    pltpu.sync_copy(src_hbm.at[wid, 0], sidx_v)
    pltpu.sync_copy(dst_hbm.at[wid, 0], didx_v)
    pltpu.async_copy(nf_hbm.at[sidx_v.at[0]], rows0, gs0)
    pltpu.async_copy(w_hbm.at[wid, 0], wrow0, ws0)
    pltpu.async_copy(attr_hbm.at[wid, 0], ax0, as0)
    pltpu.async_copy(nf_hbm.at[sidx_v.at[1]], rows1, gs1)
    pltpu.async_copy(w_hbm.at[wid, 1], wrow1, ws1)
    pltpu.async_copy(attr_hbm.at[wid, 1], ax1, as1)

    plsc.subcore_barrier()

    def _pair(i, carry):
        c0 = 2 * i
        j0 = lax.rem(c0, _KB)
        j1 = j0 + 1
        pltpu.make_async_copy(nf_hbm.at[sidx_v.at[0]], rows0, gs0).wait()
        pltpu.make_async_copy(w_hbm.at[wid, 0], wrow0, ws0).wait()
        pltpu.make_async_copy(attr_hbm.at[wid, 0], ax0, as0).wait()
        _mul_rows(rows0, wrow0, ax0)
        pltpu.async_copy(rows0, acc_sh.at[didx_v.at[j0]], ss0, add=True)
        pltpu.make_async_copy(nf_hbm.at[sidx_v.at[1]], rows1, gs1).wait()
        pltpu.make_async_copy(w_hbm.at[wid, 0], wrow1, ws1).wait()
        pltpu.make_async_copy(attr_hbm.at[wid, 0], ax1, as1).wait()
        _mul_rows(rows1, wrow1, ax1)
        pltpu.sync_copy(rows1, acc_sh.at[didx_v.at[j1]], add=True)
        pltpu.make_async_copy(rows0, acc_sh.at[didx_v.at[0]], ss0).wait()

        @pl.when(i < _NPAIR - 1)
        def _refill():
            nb = i + 1

            @pl.when(lax.rem(nb, _KB // 2) == 0)
            def _next_batch():
                b = lax.div(nb, _KB // 2)
                pltpu.sync_copy(src_hbm.at[wid, b], sidx_v)
                pltpu.sync_copy(dst_hbm.at[wid, b], didx_v)

            c0n = 2 * nb
            j0n = lax.rem(c0n, _KB)
            pltpu.async_copy(nf_hbm.at[sidx_v.at[j0n]], rows0, gs0)
            pltpu.async_copy(w_hbm.at[wid, c0n], wrow0, ws0)
            pltpu.async_copy(attr_hbm.at[wid, c0n], ax0, as0)
            pltpu.async_copy(nf_hbm.at[sidx_v.at[j0n + 1]], rows1, gs1)
            pltpu.async_copy(w_hbm.at[wid, c0n + 1], wrow1, ws1)
            pltpu.async_copy(attr_hbm.at[wid, c0n + 1], ax1, as1)

        return carry

    lax.fori_loop(0, _NPAIR, _pair, 0)
    plsc.subcore_barrier()

    base = sid * _RPS
    pltpu.sync_copy(acc_sh.at[pl.ds(base, _RPS)], out_hbm.at[cid, pl.ds(base, _RPS)])


_edge_scatter = functools.partial(
    pl.kernel,
    out_type=jax.ShapeDtypeStruct((_NC, _NP, _D), jnp.float32),
    mesh=plsc.VectorSubcoreMesh(core_axis_name="c", subcore_axis_name="s"),
    scratch_types=[
        pltpu.VMEM((_KB, _C), jnp.int32),
        pltpu.VMEM((_KB, _C), jnp.int32),
        pltpu.VMEM((_C * 16,), jnp.float32),
        pltpu.VMEM((_C * 16,), jnp.float32),
        pltpu.VMEM((_C, _D), jnp.float32),
        pltpu.VMEM((_C, _D), jnp.float32),
        pltpu.VMEM((_C, _D), jnp.float32),
        pltpu.VMEM((_C, _D), jnp.float32),
        pltpu.VMEM_SHARED((_NP, _D), jnp.float32),
        pltpu.SemaphoreType.DMA,
        pltpu.SemaphoreType.DMA,
        pltpu.SemaphoreType.DMA,
        pltpu.SemaphoreType.DMA,
        pltpu.SemaphoreType.DMA,
        pltpu.SemaphoreType.DMA,
        pltpu.SemaphoreType.DMA,
        pltpu.SemaphoreType.DMA,
    ],
)(_edge_scatter_body)


def _post_body(p0_ref, p1_ref, a_ref, wl2_ref, w3_ref, nsc_ref, out_ref):
    agg = p0_ref[...] + p1_ref[...]
    a = a_ref[...]
    acc = jnp.zeros(agg.shape, jnp.float32)
    for j in range(_A):
        acc += a[:, j:j + 1] * jnp.dot(agg, wl2_ref[j], preferred_element_type=jnp.float32)
    conv = acc * _INV_FAN
    t = jnp.dot(agg, w3_ref[...], preferred_element_type=jnp.float32)
    angle = (0.1 * _INV_FAN) * jnp.sum(t * a, axis=1, keepdims=True)
    out_ref[...] = jnp.cos(angle) * nsc_ref[...] + jnp.sin(angle) * conv


def _post(p0, p1, a, wl2_t, w3_r, nsc):
    bn = 2000
    return pl.pallas_call(
        _post_body,
        grid=(_N // bn,),
        in_specs=[
            pl.BlockSpec((bn, _D), lambda i: (i, 0)),
            pl.BlockSpec((bn, _D), lambda i: (i, 0)),
            pl.BlockSpec((bn, _A), lambda i: (i, 0)),
            pl.BlockSpec((_A, _D, _D), lambda i: (0, 0, 0)),
            pl.BlockSpec((_D, _A), lambda i: (0, 0)),
            pl.BlockSpec((bn, _D), lambda i: (i, 0)),
        ],
        out_specs=pl.BlockSpec((bn, _D), lambda i: (i, 0)),
        out_shape=jax.ShapeDtypeStruct((_N, _D), jnp.float32),
    )(p0, p1, a, wl2_t, w3_r, nsc)


def kernel(node_input, node_attr, edge_src, edge_dst, edge_attr, edge_scalars,
           num_neighbors, W_sc, W_lin1, W_fc1, W_fc2, W_lin2, W_lin3):
    wsc_t = jnp.transpose(W_sc, (1, 0, 2))
    wl1_t = jnp.transpose(W_lin1, (1, 0, 2))
    wl2_t = jnp.transpose(W_lin2, (1, 0, 2))
    w3_r = W_lin3.reshape(_D, _A)

    inv_nb = 1.0 / jnp.sqrt(jnp.asarray(num_neighbors, jnp.float32))
    wfc2_scaled = W_fc2 * ((1.0 / math.sqrt(float(_H))) * inv_nb)

    nf = _fctp(node_input, node_attr, wl1_t)
    w_scaled = _edge_mlp(edge_scalars.T, W_fc1, wfc2_scaled)

    src4 = edge_src.astype(jnp.int32).reshape(_NW, _NB, _KB, _C)
    dst4 = edge_dst.astype(jnp.int32).reshape(_NW, _NB, _KB, _C)
    attrx = jnp.broadcast_to(
        edge_attr.astype(jnp.float32), (_E, 16)).reshape(_NW, _NCHUNK, _C * 16)

    w4 = w_scaled.reshape(_NW, _NCHUNK, _C, _D)

    partials = _edge_scatter(nf, w4, src4, dst4, attrx)

    nsc = _fctp(node_input, node_attr, wsc_t)

    return _post(partials[0, :_N], partials[1, :_N], node_attr, wl2_t, w3_r, nsc)


# speedup vs baseline: 1.5280x; 1.0056x over previous
import functools
import math

import jax
import jax.numpy as jnp
from jax import lax
from jax.experimental import pallas as pl
from jax.experimental.pallas import tpu as pltpu
from jax.experimental.pallas import tpu_sc as plsc

_N = 10000
_E = 320000
_D = 128
_A = 8
_S = 16
_H = 64

_NC = 2
_NS = 16
_NW = _NC * _NS
_EPW = _E // _NW
_C = 40
_NCHUNK = _EPW // _C
_KB = 50
_NB = _NCHUNK // _KB
_NPAIR = _NCHUNK // 2
_NP = 10240
_RPS = _NP // _NS

_INV_FAN = 1.0 / math.sqrt(float(_D * _A))


def _fctp_body(x_ref, a_ref, w_ref, o_ref):
    x = x_ref[...]
    a = a_ref[...]
    acc = jnp.zeros(x.shape, jnp.float32)
    for j in range(_A):
        acc += a[:, j:j + 1] * jnp.dot(x, w_ref[j], preferred_element_type=jnp.float32)
    o_ref[...] = acc * _INV_FAN


def _fctp(x, a, w_t):
    bn = 2000
    return pl.pallas_call(
        _fctp_body,
        grid=(_N // bn,),
        in_specs=[
            pl.BlockSpec((bn, _D), lambda i: (i, 0)),
            pl.BlockSpec((bn, _A), lambda i: (i, 0)),
            pl.BlockSpec((_A, _D, _D), lambda i: (0, 0, 0)),
        ],
        out_specs=pl.BlockSpec((bn, _D), lambda i: (i, 0)),
        out_shape=jax.ShapeDtypeStruct((_N, _D), jnp.float32),
    )(x, a, w_t)


def _edge_mlp_body(est_ref, wfc1_ref, wfc2_ref, out_ref):
    est = est_ref[...]
    h = lax.dot_general(est, wfc1_ref[...], (((0,), (0,)), ((), ())),
                        preferred_element_type=jnp.float32)
    h = h * (1.0 / math.sqrt(float(_S)))
    h = h * jax.nn.sigmoid(h)
    out_ref[...] = jnp.dot(h, wfc2_ref[...], preferred_element_type=jnp.float32)


def _edge_mlp(es_t, wfc1, wfc2_scaled):
    be = 3200
    return pl.pallas_call(
        _edge_mlp_body,
        grid=(_E // be,),
        in_specs=[
            pl.BlockSpec((_S, be), lambda i: (0, i)),
            pl.BlockSpec((_S, _H), lambda i: (0, 0)),
            pl.BlockSpec((_H, _D), lambda i: (0, 0)),
        ],
        out_specs=pl.BlockSpec((be, _D), lambda i: (i, 0)),
        out_shape=jax.ShapeDtypeStruct((_E, _D), jnp.float32),
    )(es_t, wfc1, wfc2_scaled)


def _mul_rows(rows, wrow, ax):
    @plsc.parallel_loop(0, _C, unroll=4)
    def _mul(e):
        av = ax[pl.ds(e * 16, 16)]
        for k in range(_D // 16):
            sl = pl.ds(k * 16, 16)
            rows[e, sl] = rows[e, sl] * wrow[e, sl] * av


def _edge_scatter_body(nf_hbm, w_hbm, src_hbm, dst_hbm, attr_hbm, out_hbm,
                       sidx_v, didx_v, ax0, ax1, rows0, rows1, wrow0, wrow1, acc_sh,
                       gs0, gs1, ws0, ws1, ss0, ss1, as0, as1):
    cid = lax.axis_index("c")
    sid = lax.axis_index("s")
    wid = sid * _NC + cid
    sems = (gs0, gs1, ws0, ws1, ss0, ss1)

    zero16 = jnp.zeros((16,), jnp.float32)

    def _zero_row(i, carry):
        for k in range(_D // 16):
            wrow0[i, pl.ds(k * 16, 16)] = zero16
        return carry

    lax.fori_loop(0, _C, _zero_row, 0)
    zdescs = []
    for jj in range(_RPS // _C):
        zdescs.append(pltpu.async_copy(
            wrow0, acc_sh.at[pl.ds(sid * _RPS + jj * _C, _C)], sems[jj % 6]))
    for d in zdescs:
        d.wait()

    pltpu.sync_copy(src_hbm.at[wid, 0], sidx_v)
    pltpu.sync_copy(dst_hbm.at[wid, 0], didx_v)
    pltpu.async_copy(nf_hbm.at[sidx_v.at[0]], rows0, gs0)
    pltpu.async_copy(w_hbm.at[wid, 0], wrow0, ws0)
    pltpu.async_copy(attr_hbm.at[wid, 0], ax0, as0)
    pltpu.async_copy(nf_hbm.at[sidx_v.at[1]], rows1, gs1)
    pltpu.async_copy(w_hbm.at[wid, 1], wrow1, ws1)
    pltpu.async_copy(attr_hbm.at[wid, 1], ax1, as1)

    plsc.subcore_barrier()

    def _pair(i, carry):
        c0 = 2 * i
        j0 = lax.rem(c0, _KB)
        j1 = j0 + 1
        pltpu.make_async_copy(nf_hbm.at[sidx_v.at[0]], rows0, gs0).wait()
        pltpu.make_async_copy(w_hbm.at[wid, 0], wrow0, ws0).wait()
        pltpu.make_async_copy(attr_hbm.at[wid, 0], ax0, as0).wait()
        _mul_rows(rows0, wrow0, ax0)
        pltpu.async_copy(rows0, acc_sh.at[didx_v.at[j0]], ss0, add=True)
        pltpu.make_async_copy(nf_hbm.at[sidx_v.at[1]], rows1, gs1).wait()
        pltpu.make_async_copy(w_hbm.at[wid, 0], wrow1, ws1).wait()
        pltpu.make_async_copy(attr_hbm.at[wid, 0], ax1, as1).wait()
        _mul_rows(rows1, wrow1, ax1)
        pltpu.sync_copy(rows1, acc_sh.at[didx_v.at[j1]], add=True)
        pltpu.make_async_copy(rows0, acc_sh.at[didx_v.at[0]], ss0).wait()

        @pl.when(i < _NPAIR - 1)
        def _refill():
            nb = i + 1

            @pl.when(lax.rem(nb, _KB // 2) == 0)
            def _next_batch():
                b = lax.div(nb, _KB // 2)
                pltpu.sync_copy(src_hbm.at[wid, b], sidx_v)
                pltpu.sync_copy(dst_hbm.at[wid, b], didx_v)

            c0n = 2 * nb
            j0n = lax.rem(c0n, _KB)
            pltpu.async_copy(nf_hbm.at[sidx_v.at[j0n]], rows0, gs0)
            pltpu.async_copy(w_hbm.at[wid, c0n], wrow0, ws0)
            pltpu.async_copy(attr_hbm.at[wid, c0n], ax0, as0)
            pltpu.async_copy(nf_hbm.at[sidx_v.at[j0n + 1]], rows1, gs1)
            pltpu.async_copy(w_hbm.at[wid, c0n + 1], wrow1, ws1)
            pltpu.async_copy(attr_hbm.at[wid, c0n + 1], ax1, as1)

        return carry

    lax.fori_loop(0, _NPAIR, _pair, 0)
    plsc.subcore_barrier()

    base = sid * _RPS
    pltpu.sync_copy(acc_sh.at[pl.ds(base, _RPS)], out_hbm.at[cid, pl.ds(base, _RPS)])


_edge_scatter = functools.partial(
    pl.kernel,
    out_type=jax.ShapeDtypeStruct((_NC, _NP, _D), jnp.float32),
    mesh=plsc.VectorSubcoreMesh(core_axis_name="c", subcore_axis_name="s"),
    scratch_types=[
        pltpu.VMEM((_KB, _C), jnp.int32),
        pltpu.VMEM((_KB, _C), jnp.int32),
        pltpu.VMEM((_C * 16,), jnp.float32),
        pltpu.VMEM((_C * 16,), jnp.float32),
        pltpu.VMEM((_C, _D), jnp.float32),
        pltpu.VMEM((_C, _D), jnp.float32),
        pltpu.VMEM((_C, _D), jnp.float32),
        pltpu.VMEM((_C, _D), jnp.float32),
        pltpu.VMEM_SHARED((_NP, _D), jnp.float32),
        pltpu.SemaphoreType.DMA,
        pltpu.SemaphoreType.DMA,
        pltpu.SemaphoreType.DMA,
        pltpu.SemaphoreType.DMA,
        pltpu.SemaphoreType.DMA,
        pltpu.SemaphoreType.DMA,
        pltpu.SemaphoreType.DMA,
        pltpu.SemaphoreType.DMA,
    ],
)(_edge_scatter_body)


def _post_body(p0_ref, p1_ref, a_ref, wl2_ref, w3_ref, nsc_ref, out_ref):
    agg = p0_ref[...] + p1_ref[...]
    a = a_ref[...]
    acc = jnp.zeros(agg.shape, jnp.float32)
    for j in range(_A):
        acc += a[:, j:j + 1] * jnp.dot(agg, wl2_ref[j], preferred_element_type=jnp.float32)
    conv = acc * _INV_FAN
    t = jnp.dot(agg, w3_ref[...], preferred_element_type=jnp.float32)
    angle = (0.1 * _INV_FAN) * jnp.sum(t * a, axis=1, keepdims=True)
    out_ref[...] = jnp.cos(angle) * nsc_ref[...] + jnp.sin(angle) * conv


def _post(p0, p1, a, wl2_t, w3_r, nsc):
    bn = 2000
    return pl.pallas_call(
        _post_body,
        grid=(_N // bn,),
        in_specs=[
            pl.BlockSpec((bn, _D), lambda i: (i, 0)),
            pl.BlockSpec((bn, _D), lambda i: (i, 0)),
            pl.BlockSpec((bn, _A), lambda i: (i, 0)),
            pl.BlockSpec((_A, _D, _D), lambda i: (0, 0, 0)),
            pl.BlockSpec((_D, _A), lambda i: (0, 0)),
            pl.BlockSpec((bn, _D), lambda i: (i, 0)),
        ],
        out_specs=pl.BlockSpec((bn, _D), lambda i: (i, 0)),
        out_shape=jax.ShapeDtypeStruct((_N, _D), jnp.float32),
    )(p0, p1, a, wl2_t, w3_r, nsc)


def kernel(node_input, node_attr, edge_src, edge_dst, edge_attr, edge_scalars,
           num_neighbors, W_sc, W_lin1, W_fc1, W_fc2, W_lin2, W_lin3):
    wsc_t = jnp.transpose(W_sc, (1, 0, 2))
    wl1_t = jnp.transpose(W_lin1, (1, 0, 2))
    wl2_t = jnp.transpose(W_lin2, (1, 0, 2))
    w3_r = W_lin3.reshape(_D, _A)

    inv_nb = 1.0 / jnp.sqrt(jnp.asarray(num_neighbors, jnp.float32))
    wfc2_scaled = W_fc2 * ((1.0 / math.sqrt(float(_H))) * inv_nb)

    nf = _fctp(node_input, node_attr, wl1_t)
    w_scaled = _edge_mlp(edge_scalars.T, W_fc1, wfc2_scaled)

    src4 = edge_src.astype(jnp.int32).reshape(_NW, _NB, _KB, _C)
    dst4 = edge_dst.astype(jnp.int32).reshape(_NW, _NB, _KB, _C)
    attrx = jnp.broadcast_to(
        edge_attr.astype(jnp.float32), (_E, 16)).reshape(_NW, _NCHUNK, _C * 16)

    w4 = w_scaled.reshape(_NW, _NCHUNK, _C, _D)

    partials = _edge_scatter(nf, w4, src4, dst4, attrx)

    nsc = _fctp(node_input, node_attr, wsc_t)

    return _post(partials[0, :_N], partials[1, :_N], node_attr, wl2_t, w3_r, nsc)


# speedup vs baseline: 1.8952x; 1.2404x over previous
import functools
import math

import jax
import jax.numpy as jnp
from jax import lax
from jax.experimental import pallas as pl
from jax.experimental.pallas import tpu as pltpu
from jax.experimental.pallas import tpu_sc as plsc

_N = 10000
_E = 320000
_D = 128
_A = 8
_S = 16
_H = 64

_NC = 2
_NS = 16
_NW = _NC * _NS
_EPW = _E // _NW
_C = 40
_NCHUNK = _EPW // _C
_KB = 10
_NB = _NCHUNK // _KB
_NP = 10240
_RPS = _NP // _NS

_INV_FAN = 1.0 / math.sqrt(float(_D * _A))


def _fctp_body(x_ref, a_ref, w_ref, o_ref):
    x = x_ref[...]
    a = a_ref[...]
    acc = jnp.zeros(x.shape, jnp.float32)
    for j in range(_A):
        acc += a[:, j:j + 1] * jnp.dot(x, w_ref[j], preferred_element_type=jnp.float32)
    o_ref[...] = acc * _INV_FAN


def _fctp(x, a, w_t):
    bn = 2000
    return pl.pallas_call(
        _fctp_body,
        grid=(_N // bn,),
        in_specs=[
            pl.BlockSpec((bn, _D), lambda i: (i, 0)),
            pl.BlockSpec((bn, _A), lambda i: (i, 0)),
            pl.BlockSpec((_A, _D, _D), lambda i: (0, 0, 0)),
        ],
        out_specs=pl.BlockSpec((bn, _D), lambda i: (i, 0)),
        out_shape=jax.ShapeDtypeStruct((_N, _D), jnp.float32),
    )(x, a, w_t)


def _edge_mlp_body(est_ref, wfc1_ref, wfc2_ref, out_ref):
    est = est_ref[...]
    h = lax.dot_general(est, wfc1_ref[...], (((0,), (0,)), ((), ())),
                        preferred_element_type=jnp.float32)
    h = h * (1.0 / math.sqrt(float(_S)))
    h = h * jax.nn.sigmoid(h)
    out_ref[...] = jnp.dot(h, wfc2_ref[...], preferred_element_type=jnp.float32)


def _edge_mlp(es_t, wfc1, wfc2_scaled):
    be = 3200
    return pl.pallas_call(
        _edge_mlp_body,
        grid=(_E // be,),
        in_specs=[
            pl.BlockSpec((_S, be), lambda i: (0, i)),
            pl.BlockSpec((_S, _H), lambda i: (0, 0)),
            pl.BlockSpec((_H, _D), lambda i: (0, 0)),
        ],
        out_specs=pl.BlockSpec((be, _D), lambda i: (i, 0)),
        out_shape=jax.ShapeDtypeStruct((_E, _D), jnp.float32),
    )(es_t, wfc1, wfc2_scaled)


def _mul_rows(rows, wrow, ax):
    @plsc.parallel_loop(0, _C, unroll=4)
    def _mul(e):
        av = ax[pl.ds(e * 16, 16)]
        for k in range(_D // 16):
            sl = pl.ds(k * 16, 16)
            rows[e, sl] = rows[e, sl] * wrow[e, sl] * av


def _edge_scatter_body(nf_hbm, w_hbm, src_hbm, dst_hbm, attr_hbm, out_hbm,
                       sidxA, sidxB, didxA, didxB, ax0, ax1, ax2,
                       rows0, rows1, rows2, wrow0, wrow1, wrow2, acc_sh,
                       gs0, gs1, gs2, ws0, ws1, ws2, as0, as1, as2,
                       ss0, ss1, ss2, ib0, ib1):
    cid = lax.axis_index("c")
    sid = lax.axis_index("s")
    wid = sid * _NC + cid
    sidx = (sidxA, sidxB)
    didx = (didxA, didxB)
    ax = (ax0, ax1, ax2)
    rows = (rows0, rows1, rows2)
    wrow = (wrow0, wrow1, wrow2)
    gs = (gs0, gs1, gs2)
    ws = (ws0, ws1, ws2)
    asem = (as0, as1, as2)
    ss = (ss0, ss1, ss2)
    ib = (ib0, ib1)

    zero16 = jnp.zeros((16,), jnp.float32)

    def _zero_row(i, carry):
        for k in range(_D // 16):
            wrow0[i, pl.ds(k * 16, 16)] = zero16
        return carry

    lax.fori_loop(0, _C, _zero_row, 0)
    zdescs = []
    for jj in range(_RPS // _C):
        zdescs.append(pltpu.async_copy(
            wrow0, acc_sh.at[pl.ds(sid * _RPS + jj * _C, _C)], ss[jj % 3]))
    for d in zdescs:
        d.wait()

    def _fire_p(c, s, p):
        pltpu.async_copy(nf_hbm.at[sidx[p].at[lax.rem(c, _KB)]], rows[s], gs[s])
        pltpu.async_copy(w_hbm.at[wid, c], wrow[s], ws[s])
        pltpu.async_copy(attr_hbm.at[wid, c], ax[s], asem[s])

    def _fire(c, s):
        par = lax.rem(lax.div(c, _KB), 2)

        @pl.when(par == 0)
        def _f0():
            _fire_p(c, s, 0)

        @pl.when(par == 1)
        def _f1():
            _fire_p(c, s, 1)

    def _wait_chunk(s):
        pltpu.make_async_copy(nf_hbm.at[sidx[0].at[0]], rows[s], gs[s]).wait()
        pltpu.make_async_copy(w_hbm.at[wid, 0], wrow[s], ws[s]).wait()
        pltpu.make_async_copy(attr_hbm.at[wid, 0], ax[s], asem[s]).wait()

    def _drain_scatter(s):
        pltpu.make_async_copy(rows[s], acc_sh.at[didx[0].at[0]], ss[s]).wait()

    def _load_batch(b, p):
        pltpu.async_copy(src_hbm.at[wid, b], sidx[p], ib[p])
        pltpu.async_copy(dst_hbm.at[wid, b], didx[p], ib[p])

    def _wait_batch(p):
        pltpu.make_async_copy(src_hbm.at[wid, 0], sidx[p], ib[p]).wait()
        pltpu.make_async_copy(dst_hbm.at[wid, 0], didx[p], ib[p]).wait()

    pltpu.sync_copy(src_hbm.at[wid, 0], sidxA)
    pltpu.sync_copy(dst_hbm.at[wid, 0], didxA)
    _fire_p(0, 0, 0)
    _fire_p(1, 1, 0)

    plsc.subcore_barrier()

    def _step(c, s):
        _wait_chunk(s)
        _mul_rows(rows[s], wrow[s], ax[s])
        j = lax.rem(c, _KB)
        par = lax.rem(lax.div(c, _KB), 2)

        @pl.when(par == 0)
        def _sc0():
            pltpu.async_copy(rows[s], acc_sh.at[didx[0].at[j]], ss[s], add=True)

        @pl.when(par == 1)
        def _sc1():
            pltpu.async_copy(rows[s], acc_sh.at[didx[1].at[j]], ss[s], add=True)

        @pl.when(c >= 1)
        def _normal_drain():
            _drain_scatter((s + 2) % 3)

        b = lax.div(c, _KB)

        @pl.when(jnp.logical_and(j == 0, b + 1 < _NB))
        def _prefetch_batch():
            @pl.when(lax.rem(b + 1, 2) == 0)
            def _p0():
                _load_batch(b + 1, 0)

            @pl.when(lax.rem(b + 1, 2) == 1)
            def _p1():
                _load_batch(b + 1, 1)

        @pl.when(c + 2 < _NCHUNK)
        def _refill():
            @pl.when(lax.rem(c + 2, _KB) == 0)
            def _await_batch():
                par2 = lax.rem(lax.div(c + 2, _KB), 2)

                @pl.when(par2 == 0)
                def _w0():
                    _wait_batch(0)

                @pl.when(par2 == 1)
                def _w1():
                    _wait_batch(1)

            _fire(c + 2, (s + 2) % 3)

    def _iter(i, carry):
        c0 = 3 * i
        _step(c0, 0)
        _step(c0 + 1, 1)
        _step(c0 + 2, 2)
        return carry

    _nit = _NCHUNK // 3
    lax.fori_loop(0, _nit, _iter, 0)
    for t in range(_NCHUNK % 3):
        c = _nit * 3 + t
        _step(c, c % 3)
    _drain_scatter((_NCHUNK - 1) % 3)

    plsc.subcore_barrier()

    base = sid * _RPS
    pltpu.sync_copy(acc_sh.at[pl.ds(base, _RPS)], out_hbm.at[cid, pl.ds(base, _RPS)])


_edge_scatter = functools.partial(
    pl.kernel,
    out_type=jax.ShapeDtypeStruct((_NC, _NP, _D), jnp.float32),
    mesh=plsc.VectorSubcoreMesh(core_axis_name="c", subcore_axis_name="s"),
    scratch_types=[
        pltpu.VMEM((_KB, _C), jnp.int32),
        pltpu.VMEM((_KB, _C), jnp.int32),
        pltpu.VMEM((_KB, _C), jnp.int32),
        pltpu.VMEM((_KB, _C), jnp.int32),
        pltpu.VMEM((_C * 16,), jnp.float32),
        pltpu.VMEM((_C * 16,), jnp.float32),
        pltpu.VMEM((_C * 16,), jnp.float32),
        pltpu.VMEM((_C, _D), jnp.float32),
        pltpu.VMEM((_C, _D), jnp.float32),
        pltpu.VMEM((_C, _D), jnp.float32),
        pltpu.VMEM((_C, _D), jnp.float32),
        pltpu.VMEM((_C, _D), jnp.float32),
        pltpu.VMEM((_C, _D), jnp.float32),
        pltpu.VMEM_SHARED((_NP, _D), jnp.float32),
        pltpu.SemaphoreType.DMA,
        pltpu.SemaphoreType.DMA,
        pltpu.SemaphoreType.DMA,
        pltpu.SemaphoreType.DMA,
        pltpu.SemaphoreType.DMA,
        pltpu.SemaphoreType.DMA,
        pltpu.SemaphoreType.DMA,
        pltpu.SemaphoreType.DMA,
        pltpu.SemaphoreType.DMA,
        pltpu.SemaphoreType.DMA,
        pltpu.SemaphoreType.DMA,
        pltpu.SemaphoreType.DMA,
        pltpu.SemaphoreType.DMA,
        pltpu.SemaphoreType.DMA,
    ],
)(_edge_scatter_body)


def _post_body(p0_ref, p1_ref, a_ref, wl2_ref, w3_ref, nsc_ref, out_ref):
    agg = p0_ref[...] + p1_ref[...]
    a = a_ref[...]
    acc = jnp.zeros(agg.shape, jnp.float32)
    for j in range(_A):
        acc += a[:, j:j + 1] * jnp.dot(agg, wl2_ref[j], preferred_element_type=jnp.float32)
    conv = acc * _INV_FAN
    t = jnp.dot(agg, w3_ref[...], preferred_element_type=jnp.float32)
    angle = (0.1 * _INV_FAN) * jnp.sum(t * a, axis=1, keepdims=True)
    out_ref[...] = jnp.cos(angle) * nsc_ref[...] + jnp.sin(angle) * conv


def _post(p0, p1, a, wl2_t, w3_r, nsc):
    bn = 2000
    return pl.pallas_call(
        _post_body,
        grid=(_N // bn,),
        in_specs=[
            pl.BlockSpec((bn, _D), lambda i: (i, 0)),
            pl.BlockSpec((bn, _D), lambda i: (i, 0)),
            pl.BlockSpec((bn, _A), lambda i: (i, 0)),
            pl.BlockSpec((_A, _D, _D), lambda i: (0, 0, 0)),
            pl.BlockSpec((_D, _A), lambda i: (0, 0)),
            pl.BlockSpec((bn, _D), lambda i: (i, 0)),
        ],
        out_specs=pl.BlockSpec((bn, _D), lambda i: (i, 0)),
        out_shape=jax.ShapeDtypeStruct((_N, _D), jnp.float32),
    )(p0, p1, a, wl2_t, w3_r, nsc)


def kernel(node_input, node_attr, edge_src, edge_dst, edge_attr, edge_scalars,
           num_neighbors, W_sc, W_lin1, W_fc1, W_fc2, W_lin2, W_lin3):
    wsc_t = jnp.transpose(W_sc, (1, 0, 2))
    wl1_t = jnp.transpose(W_lin1, (1, 0, 2))
    wl2_t = jnp.transpose(W_lin2, (1, 0, 2))
    w3_r = W_lin3.reshape(_D, _A)

    inv_nb = 1.0 / jnp.sqrt(jnp.asarray(num_neighbors, jnp.float32))
    wfc2_scaled = W_fc2 * ((1.0 / math.sqrt(float(_H))) * inv_nb)

    nf = _fctp(node_input, node_attr, wl1_t)
    w_scaled = _edge_mlp(edge_scalars.T, W_fc1, wfc2_scaled)

    src4 = edge_src.astype(jnp.int32).reshape(_NW, _NB, _KB, _C)
    dst4 = edge_dst.astype(jnp.int32).reshape(_NW, _NB, _KB, _C)
    attrx = jnp.broadcast_to(
        edge_attr.astype(jnp.float32), (_E, 16)).reshape(_NW, _NCHUNK, _C * 16)

    w4 = w_scaled.reshape(_NW, _NCHUNK, _C, _D)

    partials = _edge_scatter(nf, w4, src4, dst4, attrx)

    nsc = _fctp(node_input, node_attr, wsc_t)

    return _post(partials[0, :_N], partials[1, :_N], node_attr, wl2_t, w3_r, nsc)


# speedup vs baseline: 1.9343x; 1.0206x over previous
import functools
import math

import jax
import jax.numpy as jnp
from jax import lax
from jax.experimental import pallas as pl
from jax.experimental.pallas import tpu as pltpu
from jax.experimental.pallas import tpu_sc as plsc

_N = 10000
_E = 320000
_D = 128
_A = 8
_S = 16
_H = 64

_NC = 2
_NS = 16
_NW = _NC * _NS
_EPW = _E // _NW
_C = 40
_NCHUNK = _EPW // _C
_KB = 10
_NB = _NCHUNK // _KB
_NP = 10240
_RPS = _NP // _NS

_INV_FAN = 1.0 / math.sqrt(float(_D * _A))


def _fctp_body(x_ref, a_ref, w_ref, o_ref):
    x = x_ref[...]
    a = a_ref[...]
    acc = jnp.zeros(x.shape, jnp.float32)
    for j in range(_A):
        acc += a[:, j:j + 1] * jnp.dot(x, w_ref[j], preferred_element_type=jnp.float32)
    o_ref[...] = acc * _INV_FAN


def _fctp(x, a, w_t):
    bn = 2000
    return pl.pallas_call(
        _fctp_body,
        grid=(_N // bn,),
        in_specs=[
            pl.BlockSpec((bn, _D), lambda i: (i, 0)),
            pl.BlockSpec((bn, _A), lambda i: (i, 0)),
            pl.BlockSpec((_A, _D, _D), lambda i: (0, 0, 0)),
        ],
        out_specs=pl.BlockSpec((bn, _D), lambda i: (i, 0)),
        out_shape=jax.ShapeDtypeStruct((_N, _D), jnp.float32),
    )(x, a, w_t)


def _edge_mlp_body(est_ref, wfc1_ref, wfc2_ref, out_ref):
    est = est_ref[...].astype(jnp.bfloat16)
    h = lax.dot_general(est, wfc1_ref[...], (((0,), (0,)), ((), ())),
                        preferred_element_type=jnp.float32)
    h = h * (1.0 / math.sqrt(float(_S)))
    h = h * jax.nn.sigmoid(h)
    out_ref[...] = jnp.dot(h.astype(jnp.bfloat16), wfc2_ref[...],
                           preferred_element_type=jnp.float32)


def _edge_mlp(es_t, wfc1, wfc2_scaled):
    be = 3200
    return pl.pallas_call(
        _edge_mlp_body,
        grid=(_E // be,),
        in_specs=[
            pl.BlockSpec((_S, be), lambda i: (0, i)),
            pl.BlockSpec((_S, _H), lambda i: (0, 0)),
            pl.BlockSpec((_H, _D), lambda i: (0, 0)),
        ],
        name="edge_mlp",
        out_specs=pl.BlockSpec((be, _D), lambda i: (i, 0)),
        out_shape=jax.ShapeDtypeStruct((_E, _D), jnp.float32),
    )(es_t, wfc1, wfc2_scaled)


def _mul_rows(rows, wrow, ax):
    @plsc.parallel_loop(0, _C, unroll=4)
    def _mul(e):
        av = ax[pl.ds(e * 16, 16)]
        for k in range(_D // 16):
            sl = pl.ds(k * 16, 16)
            rows[e, sl] = rows[e, sl] * wrow[e, sl] * av


def _edge_scatter_body(nf_hbm, w_hbm, src_hbm, dst_hbm, attr_hbm, out_hbm,
                       sidxA, sidxB, didxA, didxB, ax0, ax1, ax2,
                       rows0, rows1, rows2, wrow0, wrow1, wrow2, acc_sh,
                       gs0, gs1, gs2, ws0, ws1, ws2, as0, as1, as2,
                       ss0, ss1, ss2, ib0, ib1):
    cid = lax.axis_index("c")
    sid = lax.axis_index("s")
    wid = sid * _NC + cid
    sidx = (sidxA, sidxB)
    didx = (didxA, didxB)
    ax = (ax0, ax1, ax2)
    rows = (rows0, rows1, rows2)
    wrow = (wrow0, wrow1, wrow2)
    gs = (gs0, gs1, gs2)
    ws = (ws0, ws1, ws2)
    asem = (as0, as1, as2)
    ss = (ss0, ss1, ss2)
    ib = (ib0, ib1)

    zero16 = jnp.zeros((16,), jnp.float32)

    def _zero_row(i, carry):
        for k in range(_D // 16):
            wrow0[i, pl.ds(k * 16, 16)] = zero16
        return carry

    lax.fori_loop(0, _C, _zero_row, 0)
    zdescs = []
    for jj in range(_RPS // _C):
        zdescs.append(pltpu.async_copy(
            wrow0, acc_sh.at[pl.ds(sid * _RPS + jj * _C, _C)], ss[jj % 3]))
    for d in zdescs:
        d.wait()

    def _fire_p(c, s, p):
        pltpu.async_copy(nf_hbm.at[sidx[p].at[lax.rem(c, _KB)]], rows[s], gs[s])
        pltpu.async_copy(w_hbm.at[wid, c], wrow[s], ws[s])
        pltpu.async_copy(attr_hbm.at[wid, c], ax[s], asem[s])

    def _fire(c, s):
        par = lax.rem(lax.div(c, _KB), 2)

        @pl.when(par == 0)
        def _f0():
            _fire_p(c, s, 0)

        @pl.when(par == 1)
        def _f1():
            _fire_p(c, s, 1)

    def _wait_chunk(s):
        pltpu.make_async_copy(nf_hbm.at[sidx[0].at[0]], rows[s], gs[s]).wait()
        pltpu.make_async_copy(w_hbm.at[wid, 0], wrow[s], ws[s]).wait()
        pltpu.make_async_copy(attr_hbm.at[wid, 0], ax[s], asem[s]).wait()

    def _drain_scatter(s):
        pltpu.make_async_copy(rows[s], acc_sh.at[didx[0].at[0]], ss[s]).wait()

    def _load_batch(b, p):
        pltpu.async_copy(src_hbm.at[wid, b], sidx[p], ib[p])
        pltpu.async_copy(dst_hbm.at[wid, b], didx[p], ib[p])

    def _wait_batch(p):
        pltpu.make_async_copy(src_hbm.at[wid, 0], sidx[p], ib[p]).wait()
        pltpu.make_async_copy(dst_hbm.at[wid, 0], didx[p], ib[p]).wait()

    pltpu.sync_copy(src_hbm.at[wid, 0], sidxA)
    pltpu.sync_copy(dst_hbm.at[wid, 0], didxA)
    _fire_p(0, 0, 0)
    _fire_p(1, 1, 0)

    plsc.subcore_barrier()

    def _step(c, s):
        _wait_chunk(s)
        _mul_rows(rows[s], wrow[s], ax[s])
        j = lax.rem(c, _KB)
        par = lax.rem(lax.div(c, _KB), 2)

        @pl.when(par == 0)
        def _sc0():
            pltpu.async_copy(rows[s], acc_sh.at[didx[0].at[j]], ss[s], add=True)

        @pl.when(par == 1)
        def _sc1():
            pltpu.async_copy(rows[s], acc_sh.at[didx[1].at[j]], ss[s], add=True)

        @pl.when(c >= 1)
        def _normal_drain():
            _drain_scatter((s + 2) % 3)

        b = lax.div(c, _KB)

        @pl.when(jnp.logical_and(j == 0, b + 1 < _NB))
        def _prefetch_batch():
            @pl.when(lax.rem(b + 1, 2) == 0)
            def _p0():
                _load_batch(b + 1, 0)

            @pl.when(lax.rem(b + 1, 2) == 1)
            def _p1():
                _load_batch(b + 1, 1)

        @pl.when(c + 2 < _NCHUNK)
        def _refill():
            @pl.when(lax.rem(c + 2, _KB) == 0)
            def _await_batch():
                par2 = lax.rem(lax.div(c + 2, _KB), 2)

                @pl.when(par2 == 0)
                def _w0():
                    _wait_batch(0)

                @pl.when(par2 == 1)
                def _w1():
                    _wait_batch(1)

            _fire(c + 2, (s + 2) % 3)

    def _iter(i, carry):
        c0 = 3 * i
        _step(c0, 0)
        _step(c0 + 1, 1)
        _step(c0 + 2, 2)
        return carry

    _nit = _NCHUNK // 3
    lax.fori_loop(0, _nit, _iter, 0)
    for t in range(_NCHUNK % 3):
        c = _nit * 3 + t
        _step(c, c % 3)
    _drain_scatter((_NCHUNK - 1) % 3)

    plsc.subcore_barrier()

    base = sid * _RPS
    pltpu.sync_copy(acc_sh.at[pl.ds(base, _RPS)], out_hbm.at[cid, pl.ds(base, _RPS)])


_edge_scatter = functools.partial(
    pl.kernel,
    out_type=jax.ShapeDtypeStruct((_NC, _NP, _D), jnp.float32),
    mesh=plsc.VectorSubcoreMesh(core_axis_name="c", subcore_axis_name="s"),
    scratch_types=[
        pltpu.VMEM((_KB, _C), jnp.int32),
        pltpu.VMEM((_KB, _C), jnp.int32),
        pltpu.VMEM((_KB, _C), jnp.int32),
        pltpu.VMEM((_KB, _C), jnp.int32),
        pltpu.VMEM((_C * 16,), jnp.float32),
        pltpu.VMEM((_C * 16,), jnp.float32),
        pltpu.VMEM((_C * 16,), jnp.float32),
        pltpu.VMEM((_C, _D), jnp.float32),
        pltpu.VMEM((_C, _D), jnp.float32),
        pltpu.VMEM((_C, _D), jnp.float32),
        pltpu.VMEM((_C, _D), jnp.float32),
        pltpu.VMEM((_C, _D), jnp.float32),
        pltpu.VMEM((_C, _D), jnp.float32),
        pltpu.VMEM_SHARED((_NP, _D), jnp.float32),
        pltpu.SemaphoreType.DMA,
        pltpu.SemaphoreType.DMA,
        pltpu.SemaphoreType.DMA,
        pltpu.SemaphoreType.DMA,
        pltpu.SemaphoreType.DMA,
        pltpu.SemaphoreType.DMA,
        pltpu.SemaphoreType.DMA,
        pltpu.SemaphoreType.DMA,
        pltpu.SemaphoreType.DMA,
        pltpu.SemaphoreType.DMA,
        pltpu.SemaphoreType.DMA,
        pltpu.SemaphoreType.DMA,
        pltpu.SemaphoreType.DMA,
        pltpu.SemaphoreType.DMA,
    ],
)(_edge_scatter_body)


def _post_body(p_ref, a_ref, wl2_ref, w3_ref, nsc_ref, out_ref):
    agg = p_ref[0] + p_ref[1]
    a = a_ref[...]
    acc = jnp.zeros(agg.shape, jnp.float32)
    for j in range(_A):
        acc += a[:, j:j + 1] * jnp.dot(agg, wl2_ref[j], preferred_element_type=jnp.float32)
    conv = acc * _INV_FAN
    t = jnp.dot(agg, w3_ref[...], preferred_element_type=jnp.float32)
    angle = (0.1 * _INV_FAN) * jnp.sum(t * a, axis=1, keepdims=True)
    out_ref[...] = jnp.cos(angle) * nsc_ref[...] + jnp.sin(angle) * conv


def _post(partials, a, wl2_t, w3_r, nsc):
    bn = 2000
    return pl.pallas_call(
        _post_body,
        grid=(_N // bn,),
        in_specs=[
            pl.BlockSpec((_NC, bn, _D), lambda i: (0, i, 0)),
            pl.BlockSpec((bn, _A), lambda i: (i, 0)),
            pl.BlockSpec((_A, _D, _D), lambda i: (0, 0, 0)),
            pl.BlockSpec((_D, _A), lambda i: (0, 0)),
            pl.BlockSpec((bn, _D), lambda i: (i, 0)),
        ],
        out_specs=pl.BlockSpec((bn, _D), lambda i: (i, 0)),
        out_shape=jax.ShapeDtypeStruct((_N, _D), jnp.float32),
    )(partials, a, wl2_t, w3_r, nsc)


def kernel(node_input, node_attr, edge_src, edge_dst, edge_attr, edge_scalars,
           num_neighbors, W_sc, W_lin1, W_fc1, W_fc2, W_lin2, W_lin3):
    wsc_t = jnp.transpose(W_sc, (1, 0, 2))
    wl1_t = jnp.transpose(W_lin1, (1, 0, 2))
    wl2_t = jnp.transpose(W_lin2, (1, 0, 2))
    w3_r = W_lin3.reshape(_D, _A)

    inv_nb = 1.0 / jnp.sqrt(jnp.asarray(num_neighbors, jnp.float32))
    wfc2_scaled = (W_fc2 * ((1.0 / math.sqrt(float(_H))) * inv_nb)).astype(jnp.bfloat16)

    nf = _fctp(node_input, node_attr, wl1_t)
    w_scaled = _edge_mlp(edge_scalars.T, W_fc1.astype(jnp.bfloat16), wfc2_scaled)

    src4 = edge_src.astype(jnp.int32).reshape(_NW, _NB, _KB, _C)
    dst4 = edge_dst.astype(jnp.int32).reshape(_NW, _NB, _KB, _C)
    attrx = jnp.broadcast_to(
        edge_attr.astype(jnp.float32), (_E, 16)).reshape(_NW, _NCHUNK, _C * 16)

    w4 = w_scaled.reshape(_NW, _NCHUNK, _C, _D)

    partials = _edge_scatter(nf, w4, src4, dst4, attrx)

    nsc = _fctp(node_input, node_attr, wsc_t)

    return _post(partials, node_attr, wl2_t, w3_r, nsc)
